# baseline (speedup 1.0000x reference)
import functools

import jax
import jax.numpy as jnp
from jax import lax
from jax.experimental import pallas as pl
from jax.experimental.pallas import tpu as pltpu
from jax.experimental.pallas import tpu_sc as plsc

B = 4096
L = 8
D = 64
NLANE = 16
NCORE = 2
NSUB = 16
NW = NCORE * NSUB
BPW = B // NW
CHUNK = 128
NCH = (BPW * L) // CHUNK
DV = D // NLANE


def _tile_body(uids_hbm, rec_hbm, src_hbm, emb_hbm, out_hbm,
               uid_v, idx1, codes_rec, codes_src, ind_v, cnt_v,
               rows_v, out_v, emb0_v, sem):
    wid = lax.axis_index("s") * NCORE + lax.axis_index("c")
    base = wid * BPW
    iota = lax.broadcasted_iota(jnp.int32, (NLANE,), 0)

    pltpu.sync_copy(uids_hbm.at[pl.ds(base, BPW)], uid_v)
    pltpu.sync_copy(emb_hbm.at[pl.ds(0, 1)], emb0_v)

    def i1_body(c, carry):
        ivec = c * NLANE + iota
        uv = plsc.load_gather(uid_v, [ivec >> 3])
        idx1[c >> 3, pl.ds((c & (NCH - 1)) * NLANE, NLANE)] = (
            uv * L + (ivec & (L - 1)))
        return carry
    lax.fori_loop(0, (BPW * L) // NLANE, i1_body, 0)

    copies = []
    for j in range(NCH):
        copies.append(pltpu.async_copy(rec_hbm.at[idx1.at[j]],
                                       codes_rec.at[j], sem))
        copies.append(pltpu.async_copy(src_hbm.at[idx1.at[j]],
                                       codes_src.at[j], sem))
    for c in copies:
        c.wait()

    e0 = [emb0_v[0, pl.ds(d * NLANE, NLANE)] for d in range(DV)]

    def do_table(codes2d, col_base):
        copies = [
            pltpu.async_copy(emb_hbm.at[codes2d.at[j]],
                             rows_v.at[pl.ds(j * CHUNK, CHUNK)], sem)
            for j in range(NCH)
        ]

        def fl_body(c, carry):
            cv = codes2d[c >> 3, pl.ds((c & (NCH - 1)) * NLANE, NLANE)]
            ind_v[pl.ds(c * NLANE, NLANE)] = jnp.where(
                cv == 0, jnp.float32(1.0), jnp.float32(0.0))
            return carry
        lax.fori_loop(0, (BPW * L) // NLANE, fl_body, 0)

        def cnt_body(bc, carry):
            bvec = (bc * NLANE + iota) * L
            acc = plsc.load_gather(ind_v, [bvec])
            for l in range(1, L):
                acc = acc + plsc.load_gather(ind_v, [bvec + l])
            cnt_v[pl.ds(bc * NLANE, NLANE)] = acc
            return carry
        lax.fori_loop(0, BPW // NLANE, cnt_body, 0)

        for c in copies:
            c.wait()

        def b_body(b, carry):
            rbase = b * L
            acc = [rows_v[rbase, pl.ds(d * NLANE, NLANE)] for d in range(DV)]
            for l in range(1, L):
                for d in range(DV):
                    acc[d] = acc[d] + rows_v[rbase + l, pl.ds(d * NLANE, NLANE)]
            cw = plsc.load_gather(cnt_v, [jnp.full((NLANE,), b, jnp.int32)])
            for d in range(DV):
                out_v[b, pl.ds(col_base + d * NLANE, NLANE)] = acc[d] - cw * e0[d]
            return carry
        lax.fori_loop(0, BPW, b_body, 0)

    do_table(codes_rec, 0)
    do_table(codes_src, D)

    pltpu.sync_copy(out_v, out_hbm.at[pl.ds(base, BPW)])


@functools.partial(
    pl.kernel,
    out_type=jax.ShapeDtypeStruct((B, 2 * D), jnp.float32),
    mesh=plsc.VectorSubcoreMesh(core_axis_name="c", subcore_axis_name="s"),
    compiler_params=pltpu.CompilerParams(needs_layout_passes=False,
                                         use_tc_tiling_on_sc=False),
    scratch_types=[
        pltpu.VMEM((BPW,), jnp.int32),
        pltpu.VMEM((NCH, CHUNK), jnp.int32),
        pltpu.VMEM((NCH, CHUNK), jnp.int32),
        pltpu.VMEM((NCH, CHUNK), jnp.int32),
        pltpu.VMEM((BPW * L,), jnp.float32),
        pltpu.VMEM((BPW,), jnp.float32),
        pltpu.VMEM((BPW * L, D), jnp.float32),
        pltpu.VMEM((BPW, 2 * D), jnp.float32),
        pltpu.VMEM((1, D), jnp.float32),
        pltpu.SemaphoreType.DMA,
    ],
)
def _user_feat_sc(uids_hbm, rec_hbm, src_hbm, emb_hbm, out_hbm,
                  uid_v, idx1, codes_rec, codes_src, ind_v, cnt_v,
                  rows_v, out_v, emb0_v, sem):
    _tile_body(uids_hbm, rec_hbm, src_hbm, emb_hbm, out_hbm,
               uid_v, idx1, codes_rec, codes_src, ind_v, cnt_v,
               rows_v, out_v, emb0_v, sem)


def kernel(user_ids, user2rec_code, user2src_code, code_embedding):
    return _user_feat_sc(user_ids, user2rec_code.reshape(-1),
                         user2src_code.reshape(-1), code_embedding)
